# Initial kernel scaffold; baseline (speedup 1.0000x reference)
import jax
import jax.numpy as jnp
from jax.experimental import pallas as pl


def kernel(idxs, syn0):
    raise NotImplementedError("write your pallas kernel here")


# speedup vs baseline: 1.8669x; 1.8669x over previous
import functools

import jax
import jax.numpy as jnp
from jax import lax
from jax.experimental import pallas as pl
from jax.experimental.pallas import tpu as pltpu
from jax.experimental.pallas import tpu_sc as plsc

_D = 64
_CHUNK = 128
_NBUF = 4


def _sc_info():
    try:
        info = plsc.get_sparse_core_info()
        return info.num_cores, info.num_subcores
    except Exception:
        return 2, 16


@functools.lru_cache(maxsize=None)
def _build(B):
    NC, NS = _sc_info()
    NW = NC * NS
    assert B % (NW * _CHUNK) == 0
    b_per_w = B // NW
    nchunks = b_per_w // _CHUNK
    ngroups = nchunks // _NBUF
    assert nchunks % _NBUF == 0

    mesh = plsc.VectorSubcoreMesh(core_axis_name="c", subcore_axis_name="s")

    @functools.partial(
        pl.kernel,
        mesh=mesh,
        out_type=jax.ShapeDtypeStruct((B, _D), jnp.float32),
        scratch_types=[
            pltpu.VMEM((b_per_w,), jnp.int32),
            pltpu.VMEM((_NBUF, _CHUNK, _D), jnp.float32),
        ]
        + [pltpu.SemaphoreType.DMA] * _NBUF
        + [pltpu.SemaphoreType.DMA] * _NBUF,
        compiler_params=pltpu.CompilerParams(use_tc_tiling_on_sc=False),
    )
    def gather_kernel(idx_hbm, table_hbm, out_hbm, idx_v, rows_v, *sems):
        gsems = sems[:_NBUF]
        ssems = sems[_NBUF:]
        wid = lax.axis_index("s") * NC + lax.axis_index("c")
        base = wid * b_per_w

        pltpu.sync_copy(idx_hbm.at[pl.ds(base, b_per_w)], idx_v)

        def start_gather(j, b):
            idx_slice = idx_v.at[pl.ds(j * _CHUNK, _CHUNK)]
            pltpu.async_copy(table_hbm.at[idx_slice], rows_v.at[b], gsems[b])

        def wait_gather(j, b):
            idx_slice = idx_v.at[pl.ds(j * _CHUNK, _CHUNK)]
            pltpu.make_async_copy(
                table_hbm.at[idx_slice], rows_v.at[b], gsems[b]
            ).wait()

        def start_store(j, b):
            dst = out_hbm.at[pl.ds(base + j * _CHUNK, _CHUNK)]
            pltpu.async_copy(rows_v.at[b], dst, ssems[b])

        def wait_store(j, b):
            dst = out_hbm.at[pl.ds(base + j * _CHUNK, _CHUNK)]
            pltpu.make_async_copy(rows_v.at[b], dst, ssems[b]).wait()

        for b in range(_NBUF):
            start_gather(b, b)

        def group_body(g, _):
            for b in range(_NBUF):
                j = g * _NBUF + b
                wait_gather(j, b)
                start_store(j, b)
                wait_store(j, b)
                start_gather(j + _NBUF, b)
            return 0

        lax.fori_loop(0, ngroups - 1, group_body, 0)

        for b in range(_NBUF):
            j = (ngroups - 1) * _NBUF + b
            wait_gather(j, b)
            start_store(j, b)
        for b in range(_NBUF):
            j = (ngroups - 1) * _NBUF + b
            wait_store(j, b)

    return gather_kernel


def kernel(idxs, syn0):
    B = idxs.shape[0] * idxs.shape[1]
    flat = idxs.reshape(-1).astype(jnp.int32)
    out = _build(B)(flat, syn0)
    return out.reshape(*idxs.shape, syn0.shape[1])


# speedup vs baseline: 1.8672x; 1.0002x over previous
import functools

import jax
import jax.numpy as jnp
from jax import lax
from jax.experimental import pallas as pl
from jax.experimental.pallas import tpu as pltpu
from jax.experimental.pallas import tpu_sc as plsc

_D = 64
_CHUNK = 256
_NBUF = 4


def _sc_info():
    try:
        info = plsc.get_sparse_core_info()
        return info.num_cores, info.num_subcores
    except Exception:
        return 2, 16


@functools.lru_cache(maxsize=None)
def _build(B):
    NC, NS = _sc_info()
    NW = NC * NS
    assert B % (NW * _CHUNK) == 0
    b_per_w = B // NW
    nchunks = b_per_w // _CHUNK
    ngroups = nchunks // _NBUF
    assert nchunks % _NBUF == 0

    mesh = plsc.VectorSubcoreMesh(core_axis_name="c", subcore_axis_name="s")

    @functools.partial(
        pl.kernel,
        mesh=mesh,
        out_type=jax.ShapeDtypeStruct((B, _D), jnp.float32),
        scratch_types=[
            pltpu.VMEM((b_per_w,), jnp.int32),
            pltpu.VMEM((_NBUF, _CHUNK, _D), jnp.float32),
        ]
        + [pltpu.SemaphoreType.DMA] * _NBUF
        + [pltpu.SemaphoreType.DMA] * _NBUF,
        compiler_params=pltpu.CompilerParams(use_tc_tiling_on_sc=False),
    )
    def gather_kernel(idx_hbm, table_hbm, out_hbm, idx_v, rows_v, *sems):
        gsems = sems[:_NBUF]
        ssems = sems[_NBUF:]
        wid = lax.axis_index("s") * NC + lax.axis_index("c")
        base = wid * b_per_w

        pltpu.sync_copy(idx_hbm.at[pl.ds(base, b_per_w)], idx_v)

        def start_gather(j, b):
            idx_slice = idx_v.at[pl.ds(j * _CHUNK, _CHUNK)]
            pltpu.async_copy(table_hbm.at[idx_slice], rows_v.at[b], gsems[b])

        def wait_gather(j, b):
            idx_slice = idx_v.at[pl.ds(j * _CHUNK, _CHUNK)]
            pltpu.make_async_copy(
                table_hbm.at[idx_slice], rows_v.at[b], gsems[b]
            ).wait()

        def start_store(j, b):
            dst = out_hbm.at[pl.ds(base + j * _CHUNK, _CHUNK)]
            pltpu.async_copy(rows_v.at[b], dst, ssems[b])

        def wait_store(j, b):
            dst = out_hbm.at[pl.ds(base + j * _CHUNK, _CHUNK)]
            pltpu.make_async_copy(rows_v.at[b], dst, ssems[b]).wait()

        for b in range(_NBUF):
            start_gather(b, b)

        def group_body(g, _):
            for b in range(_NBUF):
                j = g * _NBUF + b
                wait_gather(j, b)
                start_store(j, b)
                wait_store(j, b)
                start_gather(j + _NBUF, b)
            return 0

        lax.fori_loop(0, ngroups - 1, group_body, 0)

        for b in range(_NBUF):
            j = (ngroups - 1) * _NBUF + b
            wait_gather(j, b)
            start_store(j, b)
        for b in range(_NBUF):
            j = (ngroups - 1) * _NBUF + b
            wait_store(j, b)

    return gather_kernel


def kernel(idxs, syn0):
    B = idxs.shape[0] * idxs.shape[1]
    flat = idxs.reshape(-1).astype(jnp.int32)
    out = _build(B)(flat, syn0)
    return out.reshape(*idxs.shape, syn0.shape[1])
